# baseline (speedup 1.0000x reference)
import functools

import jax
import jax.numpy as jnp
from jax import lax
from jax.experimental import pallas as pl
from jax.experimental.pallas import tpu as pltpu
from jax.experimental.pallas import tpu_sc as plsc

BATCH = 4096
MAX_LEN = 200
EMBED = 64
NC, NS, LANES = 2, 16, 16
NW = NC * NS
BPW = BATCH // NW
CA, CB = 104, 96
NCH = EMBED // LANES


def _encoder(x_hbm, table_hbm, f_hbm, out_hbm,
             idx_v, f_v, rows0, rows1, out_v, sem0, sem1):
    wid = lax.axis_index("s") * NC + lax.axis_index("c")
    base = wid * BPW

    pltpu.sync_copy(x_hbm.at[pl.ds(base, BPW)], idx_v)
    pltpu.sync_copy(f_hbm, f_v)

    def fire(b, rows, sem):
        pltpu.make_async_copy(
            table_hbm.at[idx_v.at[b, pl.ds(0, CA)]],
            rows.at[pl.ds(0, CA)], sem).start()
        pltpu.make_async_copy(
            table_hbm.at[idx_v.at[b, pl.ds(CA, CB)]],
            rows.at[pl.ds(CA, CB)], sem).start()

    def drain(rows, sem):
        pltpu.make_async_copy(
            table_hbm.at[idx_v.at[0, pl.ds(0, CA)]],
            rows.at[pl.ds(0, CA)], sem).wait()
        pltpu.make_async_copy(
            table_hbm.at[idx_v.at[0, pl.ds(CA, CB)]],
            rows.at[pl.ds(CA, CB)], sem).wait()

    def accumulate(b, rows):
        U = 8

        def body(i, acc):
            l0 = i * U
            for u in range(U):
                acc = tuple(
                    acc[c] + rows[l0 + u, pl.ds(c * LANES, LANES)]
                    * f_v[l0 + u, pl.ds(c * LANES, LANES)]
                    for c in range(NCH))
            return acc
        acc = lax.fori_loop(
            0, MAX_LEN // U, body,
            tuple(jnp.zeros((LANES,), jnp.float32) for _ in range(NCH)))
        for c in range(NCH):
            out_v[b, pl.ds(c * LANES, LANES)] = acc[c]

    fire(0, rows0, sem0)
    fire(1, rows1, sem1)

    def gbody(g, carry):
        drain(rows0, sem0)
        accumulate(2 * g, rows0)

        @pl.when(g < BPW // 2 - 1)
        def _():
            fire(2 * g + 2, rows0, sem0)

        drain(rows1, sem1)
        accumulate(2 * g + 1, rows1)

        @pl.when(g < BPW // 2 - 1)
        def _():
            fire(2 * g + 3, rows1, sem1)

        return carry

    lax.fori_loop(0, BPW // 2, gbody, 0)

    pltpu.sync_copy(out_v, out_hbm.at[pl.ds(base, BPW)])


_mesh = plsc.VectorSubcoreMesh(core_axis_name="c", subcore_axis_name="s")

_enc = functools.partial(
    pl.kernel, mesh=_mesh,
    compiler_params=pltpu.CompilerParams(use_tc_tiling_on_sc=False),
    out_type=jax.ShapeDtypeStruct((BATCH, EMBED), jnp.float32),
    scratch_types=[
        pltpu.VMEM((BPW, MAX_LEN), jnp.int32),
        pltpu.VMEM((MAX_LEN, EMBED), jnp.float32),
        pltpu.VMEM((MAX_LEN, EMBED), jnp.float32),
        pltpu.VMEM((MAX_LEN, EMBED), jnp.float32),
        pltpu.VMEM((BPW, EMBED), jnp.float32),
        pltpu.SemaphoreType.DMA,
        pltpu.SemaphoreType.DMA,
    ],
)(_encoder)


@jax.jit
def kernel(x, table, f):
    return _enc(x.astype(jnp.int32), table, f)


# speedup vs baseline: 1.0186x; 1.0186x over previous
import functools

import jax
import jax.numpy as jnp
from jax import lax
from jax.experimental import pallas as pl
from jax.experimental.pallas import tpu as pltpu
from jax.experimental.pallas import tpu_sc as plsc

BATCH = 4096
MAX_LEN = 200
EMBED = 64
NC, NS, LANES = 2, 16, 16
NW = NC * NS
BPW = BATCH // NW
CA, CB = 104, 96
NCH = EMBED // LANES


NB = 4


def _encoder(x_hbm, table_hbm, f_hbm, out_hbm,
             idx_v, f_v, rows0, rows1, rows2, rows3, out_v,
             sem0, sem1, sem2, sem3):
    bufs = (rows0, rows1, rows2, rows3)
    sems = (sem0, sem1, sem2, sem3)
    wid = lax.axis_index("s") * NC + lax.axis_index("c")
    base = wid * BPW

    pltpu.sync_copy(x_hbm.at[pl.ds(base, BPW)], idx_v)
    pltpu.sync_copy(f_hbm, f_v)

    def fire(b, rows, sem):
        pltpu.make_async_copy(
            table_hbm.at[idx_v.at[b, pl.ds(0, CA)]],
            rows.at[pl.ds(0, CA)], sem).start()
        pltpu.make_async_copy(
            table_hbm.at[idx_v.at[b, pl.ds(CA, CB)]],
            rows.at[pl.ds(CA, CB)], sem).start()

    def drain(rows, sem):
        pltpu.make_async_copy(
            table_hbm.at[idx_v.at[0, pl.ds(0, CA)]],
            rows.at[pl.ds(0, CA)], sem).wait()
        pltpu.make_async_copy(
            table_hbm.at[idx_v.at[0, pl.ds(CA, CB)]],
            rows.at[pl.ds(CA, CB)], sem).wait()

    def accumulate(b, rows):
        U = 8

        def body(i, acc):
            l0 = i * U
            for u in range(U):
                acc = tuple(
                    acc[c] + rows[l0 + u, pl.ds(c * LANES, LANES)]
                    * f_v[l0 + u, pl.ds(c * LANES, LANES)]
                    for c in range(NCH))
            return acc
        acc = lax.fori_loop(
            0, MAX_LEN // U, body,
            tuple(jnp.zeros((LANES,), jnp.float32) for _ in range(NCH)))
        for c in range(NCH):
            out_v[b, pl.ds(c * LANES, LANES)] = acc[c]

    for j in range(NB - 1):
        fire(j, bufs[j], sems[j])

    def gbody(g, carry):
        for j in range(NB):
            b = NB * g + j
            drain(bufs[j], sems[j])
            accumulate(b, bufs[j])
            jn = (j + NB - 1) % NB

            @pl.when(b + NB - 1 < BPW)
            def _():
                fire(b + NB - 1, bufs[jn], sems[jn])

        return carry

    lax.fori_loop(0, BPW // NB, gbody, 0)

    pltpu.sync_copy(out_v, out_hbm.at[pl.ds(base, BPW)])


_mesh = plsc.VectorSubcoreMesh(core_axis_name="c", subcore_axis_name="s")

_enc = functools.partial(
    pl.kernel, mesh=_mesh,
    compiler_params=pltpu.CompilerParams(use_tc_tiling_on_sc=False),
    out_type=jax.ShapeDtypeStruct((BATCH, EMBED), jnp.float32),
    scratch_types=[
        pltpu.VMEM((BPW, MAX_LEN), jnp.int32),
        pltpu.VMEM((MAX_LEN, EMBED), jnp.float32),
        pltpu.VMEM((MAX_LEN, EMBED), jnp.float32),
        pltpu.VMEM((MAX_LEN, EMBED), jnp.float32),
        pltpu.VMEM((MAX_LEN, EMBED), jnp.float32),
        pltpu.VMEM((MAX_LEN, EMBED), jnp.float32),
        pltpu.VMEM((BPW, EMBED), jnp.float32),
        pltpu.SemaphoreType.DMA,
        pltpu.SemaphoreType.DMA,
        pltpu.SemaphoreType.DMA,
        pltpu.SemaphoreType.DMA,
    ],
)(_encoder)


@jax.jit
def kernel(x, table, f):
    return _enc(x.astype(jnp.int32), table, f)
